# baseline (speedup 1.0000x reference)
import functools

import jax
import jax.numpy as jnp
from jax import lax
from jax.experimental import pallas as pl
from jax.experimental.pallas import tpu as pltpu
from jax.experimental.pallas import tpu_sc as plsc

_R = 400


def kernel(x, pos_emb):
    B, L = x.shape
    V, D = pos_emb.shape
    NP = B * L // 2
    NW = 32
    C = NP // (NW * _R)
    JP = L // 2

    xf = x.reshape(NP, 2)
    xe = xf[:, 0]
    xo = xf[:, 1]

    row0 = pos_emb[0]
    ev = pos_emb[1:L + 1:2]
    od = pos_emb[2:L + 2:2]
    z = jnp.broadcast_to(row0, (JP, D))
    left = jnp.stack([z, z, ev, ev], axis=1)
    right = jnp.stack([z, od, z, od], axis=1)
    pairtab = jnp.concatenate([left, right], axis=-1).reshape(JP * 4, 2 * D)

    mesh = plsc.VectorSubcoreMesh(core_axis_name="c", subcore_axis_name="s")

    @functools.partial(
        pl.kernel,
        mesh=mesh,
        out_type=jax.ShapeDtypeStruct((NP, 2 * D), jnp.float32),
        scratch_types=[
            pltpu.VMEM((_R,), jnp.int32),
            pltpu.VMEM((_R,), jnp.int32),
            pltpu.VMEM((_R,), jnp.int32),
            pltpu.VMEM((_R, 2 * D), jnp.float32),
            pltpu.SemaphoreType.DMA,
        ],
    )
    def sc_kern(xe_hbm, xo_hbm, tab_hbm, out_hbm, xe_v, xo_v, idx_v, rows_v, sem):
        wid = lax.axis_index("s") * 2 + lax.axis_index("c")

        def chunk(c, carry):
            base = (wid * C + c) * _R
            pltpu.sync_copy(xe_hbm.at[pl.ds(base, _R)], xe_v)
            pltpu.sync_copy(xo_hbm.at[pl.ds(base, _R)], xo_v)
            lanes = lax.iota(jnp.int32, 16)
            for r in range(_R // 16):
                sl = pl.ds(r * 16, 16)
                j4 = lax.rem(lanes + (r * 16), JP) * 4
                me = jnp.where(xe_v[sl] != 0, 2, 0)
                mo = jnp.where(xo_v[sl] != 0, 1, 0)
                idx_v[sl] = j4 + me + mo
            pltpu.async_copy(tab_hbm.at[idx_v], rows_v, sem).wait()
            pltpu.sync_copy(rows_v, out_hbm.at[pl.ds(base, _R)])
            return carry

        lax.fori_loop(0, C, chunk, 0)

    out = sc_kern(xe, xo, pairtab)
    return out.reshape(B, L, D)


# speedup vs baseline: 1.5386x; 1.5386x over previous
import functools

import jax
import jax.numpy as jnp
from jax import lax
from jax.experimental import pallas as pl
from jax.experimental.pallas import tpu as pltpu
from jax.experimental.pallas import tpu_sc as plsc

_R = 400


def kernel(x, pos_emb):
    B, L = x.shape
    V, D = pos_emb.shape
    N = B * L
    NW = 32
    C = N // (NW * _R)
    G = _R // 16

    xf = x.reshape(N)
    bodyblk = pos_emb[1:L + 1].reshape(L * D)
    row0 = pos_emb[0]

    mesh = plsc.VectorSubcoreMesh(core_axis_name="c", subcore_axis_name="s")

    @functools.partial(
        pl.kernel,
        mesh=mesh,
        out_type=jax.ShapeDtypeStruct((N * D,), jnp.float32),
        scratch_types=[
            pltpu.VMEM((L * D,), jnp.float32),
            pltpu.VMEM((D,), jnp.float32),
            pltpu.VMEM((_R,), jnp.int32),
            pltpu.VMEM((_R,), jnp.int32),
            pltpu.VMEM((_R * D,), jnp.float32),
            pltpu.VMEM((_R * D,), jnp.float32),
            pltpu.SemaphoreType.DMA,
            pltpu.SemaphoreType.DMA,
        ],
    )
    def sc_kern(xf_hbm, body_hbm, row0_hbm, out_hbm,
                tmpl_v, row0_v, x0_v, x1_v, rows0_v, rows1_v, sem0, sem1):
        wid = lax.axis_index("s") * 2 + lax.axis_index("c")
        xbufs = (x0_v, x1_v)
        rbufs = (rows0_v, rows1_v)
        sems = (sem0, sem1)

        pltpu.sync_copy(body_hbm, tmpl_v)
        pltpu.sync_copy(row0_hbm, row0_v)
        for rv in rbufs:
            for k in range(_R // L):
                pltpu.sync_copy(body_hbm, rv.at[pl.ds(k * L * D, L * D)])

        def patch(rv, x_v, target_tmpl):
            lanes = lax.iota(jnp.int32, 16)
            for g in range(G):
                xv = x_v[pl.ds(g * 16, 16)]
                bitv = jnp.where(xv == 0, lax.shift_left(1, lanes), 0)
                bits = bitv[0]
                for i in range(1, 16):
                    bits = bits | bitv[i]

                @pl.when(bits != 0)
                def _():
                    def fix(i, carry):
                        line = g * 16 + i

                        @pl.when(lax.shift_right_logical(bits, i) & 1 != 0)
                        def _():
                            for c in range(D // 16):
                                dst = pl.ds(line * D + c * 16, 16)
                                if target_tmpl:
                                    src = pl.ds(
                                        lax.rem(line, L) * D + c * 16, 16)
                                    rv[dst] = tmpl_v[src]
                                else:
                                    rv[dst] = row0_v[pl.ds(c * 16, 16)]
                        return carry

                    lax.fori_loop(0, 16, fix, 0)

        def run_chunk(c, b, not_first):
            base = (wid * C + c) * _R

            @pl.when(not_first)
            def _():
                pltpu.make_async_copy(
                    rbufs[b], out_hbm.at[pl.ds(0, _R * D)], sems[b]).wait()
                patch(rbufs[b], xbufs[b], True)

            pltpu.sync_copy(xf_hbm.at[pl.ds(base, _R)], xbufs[b])
            patch(rbufs[b], xbufs[b], False)
            pltpu.async_copy(
                rbufs[b], out_hbm.at[pl.ds(base * D, _R * D)], sems[b])

        def pair(g, carry):
            run_chunk(2 * g, 0, g > 0)
            run_chunk(2 * g + 1, 1, g > 0)
            return carry

        lax.fori_loop(0, C // 2, pair, 0)
        for b in range(2):
            pltpu.make_async_copy(
                rbufs[b], out_hbm.at[pl.ds(0, _R * D)], sems[b]).wait()

    out = sc_kern(xf, bodyblk, row0)
    return out.reshape(B, L, D)


# speedup vs baseline: 2.2266x; 1.4471x over previous
import functools

import jax
import jax.numpy as jnp
from jax import lax
from jax.experimental import pallas as pl
from jax.experimental.pallas import tpu as pltpu
from jax.experimental.pallas import tpu_sc as plsc

_R = 400


def kernel(x, pos_emb):
    B, L = x.shape
    V, D = pos_emb.shape
    N = B * L
    NW = 32
    C = N // (NW * _R)
    G = _R // 16

    xf = x.reshape(N)
    bodyblk = pos_emb[1:L + 1]
    row0 = pos_emb[0]

    mesh = plsc.VectorSubcoreMesh(core_axis_name="c", subcore_axis_name="s")

    @functools.partial(
        pl.kernel,
        mesh=mesh,
        out_type=jax.ShapeDtypeStruct((N, D), jnp.float32),
        scratch_types=[
            pltpu.VMEM((L, D), jnp.float32),
            pltpu.VMEM((D,), jnp.float32),
            pltpu.VMEM((_R,), jnp.int32),
            pltpu.VMEM((_R,), jnp.int32),
            pltpu.VMEM((_R, D), jnp.float32),
            pltpu.VMEM((_R, D), jnp.float32),
            pltpu.SemaphoreType.DMA,
            pltpu.SemaphoreType.DMA,
        ],
    )
    def sc_kern(xf_hbm, body_hbm, row0_hbm, out_hbm,
                tmpl_v, row0_v, x0_v, x1_v, rows0_v, rows1_v, sem0, sem1):
        wid = lax.axis_index("s") * 2 + lax.axis_index("c")
        xbufs = (x0_v, x1_v)
        rbufs = (rows0_v, rows1_v)
        sems = (sem0, sem1)

        pltpu.sync_copy(body_hbm, tmpl_v)
        pltpu.sync_copy(row0_hbm, row0_v)
        for rv in rbufs:
            for k in range(_R // L):
                pltpu.sync_copy(body_hbm, rv.at[pl.ds(k * L, L)])

        def patch(rv, x_v, target_tmpl):
            lanes = lax.iota(jnp.int32, 16)
            for g in range(G):
                xv = x_v[pl.ds(g * 16, 16)]
                bitv = jnp.where(xv == 0, lax.shift_left(1, lanes), 0)
                bits = bitv[0]
                for i in range(1, 16):
                    bits = bits | bitv[i]

                @pl.when(bits != 0)
                def _():
                    def fix(i, carry):
                        line = g * 16 + i

                        @pl.when(lax.shift_right_logical(bits, i) & 1 != 0)
                        def _():
                            for c in range(D // 16):
                                sl = pl.ds(c * 16, 16)
                                if target_tmpl:
                                    rv[line, sl] = tmpl_v[lax.rem(line, L), sl]
                                else:
                                    rv[line, sl] = row0_v[sl]
                        return carry

                    lax.fori_loop(0, 16, fix, 0)

        def run_chunk(c, b, not_first):
            base = (wid * C + c) * _R

            @pl.when(not_first)
            def _():
                pltpu.make_async_copy(
                    rbufs[b], out_hbm.at[pl.ds(0, _R)], sems[b]).wait()
                patch(rbufs[b], xbufs[b], True)

            pltpu.sync_copy(xf_hbm.at[pl.ds(base, _R)], xbufs[b])
            patch(rbufs[b], xbufs[b], False)
            pltpu.async_copy(
                rbufs[b], out_hbm.at[pl.ds(base, _R)], sems[b])

        def pair(g, carry):
            run_chunk(2 * g, 0, g > 0)
            run_chunk(2 * g + 1, 1, g > 0)
            return carry

        lax.fori_loop(0, C // 2, pair, 0)
        for b in range(2):
            pltpu.make_async_copy(
                rbufs[b], out_hbm.at[pl.ds(0, _R)], sems[b]).wait()

    out = sc_kern(xf, bodyblk, row0)
    return out.reshape(B, L, D)


# speedup vs baseline: 2.2412x; 1.0066x over previous
import functools

import jax
import jax.numpy as jnp
from jax import lax
from jax.experimental import pallas as pl
from jax.experimental.pallas import tpu as pltpu
from jax.experimental.pallas import tpu_sc as plsc

_R = 400


def kernel(x, pos_emb):
    B, L = x.shape
    V, D = pos_emb.shape
    N = B * L
    NW = 32
    C = N // (NW * _R)
    G = _R // 16

    xf = x.reshape(N)
    bodyblk = pos_emb[1:L + 1]
    row0 = pos_emb[0]

    mesh = plsc.VectorSubcoreMesh(core_axis_name="c", subcore_axis_name="s")

    @functools.partial(
        pl.kernel,
        mesh=mesh,
        out_type=jax.ShapeDtypeStruct((N, D), jnp.float32),
        compiler_params=pltpu.CompilerParams(use_tc_tiling_on_sc=True),
        scratch_types=[
            pltpu.VMEM((L, D), jnp.float32),
            pltpu.VMEM((D,), jnp.float32),
            pltpu.VMEM((_R,), jnp.int32),
            pltpu.VMEM((_R,), jnp.int32),
            pltpu.VMEM((_R, D), jnp.float32),
            pltpu.VMEM((_R, D), jnp.float32),
            pltpu.SemaphoreType.DMA,
            pltpu.SemaphoreType.DMA,
        ],
    )
    def sc_kern(xf_hbm, body_hbm, row0_hbm, out_hbm,
                tmpl_v, row0_v, x0_v, x1_v, rows0_v, rows1_v, sem0, sem1):
        wid = lax.axis_index("s") * 2 + lax.axis_index("c")
        xbufs = (x0_v, x1_v)
        rbufs = (rows0_v, rows1_v)
        sems = (sem0, sem1)

        pltpu.sync_copy(body_hbm, tmpl_v)
        pltpu.sync_copy(row0_hbm, row0_v)
        for rv in rbufs:
            for k in range(_R // L):
                pltpu.sync_copy(body_hbm, rv.at[pl.ds(k * L, L)])

        def patch(rv, x_v, target_tmpl):
            lanes = lax.iota(jnp.int32, 16)
            for g in range(G):
                xv = x_v[pl.ds(g * 16, 16)]
                bitv = jnp.where(xv == 0, lax.shift_left(1, lanes), 0)
                bits = bitv[0]
                for i in range(1, 16):
                    bits = bits | bitv[i]

                @pl.when(bits != 0)
                def _():
                    def fix(i, carry):
                        line = g * 16 + i

                        @pl.when(lax.shift_right_logical(bits, i) & 1 != 0)
                        def _():
                            for c in range(D // 16):
                                sl = pl.ds(c * 16, 16)
                                if target_tmpl:
                                    rv[line, sl] = tmpl_v[lax.rem(line, L), sl]
                                else:
                                    rv[line, sl] = row0_v[sl]
                        return carry

                    lax.fori_loop(0, 16, fix, 0)

        def run_chunk(c, b, not_first):
            base = (wid * C + c) * _R

            @pl.when(not_first)
            def _():
                pltpu.make_async_copy(
                    rbufs[b], out_hbm.at[pl.ds(0, _R)], sems[b]).wait()
                patch(rbufs[b], xbufs[b], True)

            pltpu.sync_copy(xf_hbm.at[pl.ds(base, _R)], xbufs[b])
            patch(rbufs[b], xbufs[b], False)
            pltpu.async_copy(
                rbufs[b], out_hbm.at[pl.ds(base, _R)], sems[b])

        def pair(g, carry):
            run_chunk(2 * g, 0, g > 0)
            run_chunk(2 * g + 1, 1, g > 0)
            return carry

        lax.fori_loop(0, C // 2, pair, 0)
        for b in range(2):
            pltpu.make_async_copy(
                rbufs[b], out_hbm.at[pl.ds(0, _R)], sems[b]).wait()

    out = sc_kern(xf, bodyblk, row0)
    return out.reshape(B, L, D)


# speedup vs baseline: 4.0322x; 1.7991x over previous
import functools

import jax
import jax.numpy as jnp
from jax import lax
from jax.experimental import pallas as pl
from jax.experimental.pallas import tpu as pltpu
from jax.experimental.pallas import tpu_sc as plsc

_R = 400


def kernel(x, pos_emb):
    B, L = x.shape
    V, D = pos_emb.shape
    N = B * L
    NW = 32
    C = N // (NW * _R)
    G = _R // 16

    xf = x.reshape(N)
    bodyblk = pos_emb[1:L + 1]
    row0 = pos_emb[0]

    mesh = plsc.VectorSubcoreMesh(core_axis_name="c", subcore_axis_name="s")

    @functools.partial(
        pl.kernel,
        mesh=mesh,
        out_type=jax.ShapeDtypeStruct((N, D), jnp.float32),
        compiler_params=pltpu.CompilerParams(use_tc_tiling_on_sc=True),
        scratch_types=[
            pltpu.VMEM((L, D), jnp.float32),
            pltpu.VMEM((D,), jnp.float32),
            pltpu.VMEM((_R,), jnp.int32),
            pltpu.VMEM((_R,), jnp.int32),
            pltpu.VMEM((_R, D), jnp.float32),
            pltpu.VMEM((_R, D), jnp.float32),
            pltpu.SemaphoreType.DMA,
            pltpu.SemaphoreType.DMA,
        ],
    )
    def sc_kern(xf_hbm, body_hbm, row0_hbm, out_hbm,
                tmpl_v, row0_v, x0_v, x1_v, rows0_v, rows1_v, sem0, sem1):
        wid = lax.axis_index("s") * 2 + lax.axis_index("c")
        xbufs = (x0_v, x1_v)
        rbufs = (rows0_v, rows1_v)
        sems = (sem0, sem1)

        pltpu.sync_copy(body_hbm, tmpl_v)
        pltpu.sync_copy(row0_hbm, row0_v)
        for rv in rbufs:
            for k in range(_R // L):
                pltpu.sync_copy(body_hbm, rv.at[pl.ds(k * L, L)])

        def patch(rv, x_v, target_tmpl):
            lanes = lax.iota(jnp.int32, 16)
            for g in range(G):
                xv = x_v[pl.ds(g * 16, 16)]
                bitv = jnp.where(xv == 0, lax.shift_left(1, lanes), 0)
                bits = bitv[0]
                for i in range(1, 16):
                    bits = bits | bitv[i]

                @pl.when(bits != 0)
                def _():
                    def fix(i, carry):
                        line = g * 16 + i

                        @pl.when(lax.shift_right_logical(bits, i) & 1 != 0)
                        def _():
                            for c in range(D // 16):
                                sl = pl.ds(c * 16, 16)
                                if target_tmpl:
                                    rv[line, sl] = tmpl_v[lax.rem(line, L), sl]
                                else:
                                    rv[line, sl] = row0_v[sl]
                        return carry

                    lax.fori_loop(0, 16, fix, 0)

        def has_zero(x_v):
            acc = x_v[pl.ds(0, 16)]
            for g in range(1, G):
                acc = jnp.minimum(acc, x_v[pl.ds(g * 16, 16)])
            m = acc[0]
            for i in range(1, 16):
                m = jnp.minimum(m, acc[i])
            return m == 0

        def run_chunk(c, b, not_first, dirty_prev):
            base = (wid * C + c) * _R

            @pl.when(not_first)
            def _():
                pltpu.make_async_copy(
                    rbufs[b], out_hbm.at[pl.ds(0, _R)], sems[b]).wait()

            @pl.when(jnp.logical_and(not_first, dirty_prev))
            def _():
                patch(rbufs[b], xbufs[b], True)

            pltpu.sync_copy(xf_hbm.at[pl.ds(base, _R)], xbufs[b])
            dirty = has_zero(xbufs[b])

            @pl.when(dirty)
            def _():
                patch(rbufs[b], xbufs[b], False)

            pltpu.async_copy(
                rbufs[b], out_hbm.at[pl.ds(base, _R)], sems[b])
            return dirty

        def pair(g, carry):
            d0, d1 = carry
            d0 = run_chunk(2 * g, 0, g > 0, d0)
            d1 = run_chunk(2 * g + 1, 1, g > 0, d1)
            return d0, d1

        lax.fori_loop(0, C // 2, pair, (jnp.bool_(False), jnp.bool_(False)))
        for b in range(2):
            pltpu.make_async_copy(
                rbufs[b], out_hbm.at[pl.ds(0, _R)], sems[b]).wait()

    out = sc_kern(xf, bodyblk, row0)
    return out.reshape(B, L, D)
